# Initial kernel scaffold; baseline (speedup 1.0000x reference)
import jax
import jax.numpy as jnp
from jax.experimental import pallas as pl


def kernel(text, emb, W1, b1, W2, b2, W3, b3):
    raise NotImplementedError("write your pallas kernel here")


# speedup vs baseline: 2.6609x; 2.6609x over previous
import functools

import jax
import jax.numpy as jnp
from jax import lax
from jax.experimental import pallas as pl
from jax.experimental.pallas import tpu as pltpu
from jax.experimental.pallas import tpu_sc as plsc

VOCAB = 1000000
EMBED = 64
B = 16384
L = 50

try:
    _info = plsc.get_sparse_core_info()
    _NC, _NS = _info.num_cores, _info.num_subcores
except Exception:
    _NC, _NS = 2, 16
_NW = _NC * _NS
_BPW = B // _NW


def _bag_body(textT_hbm, emb_hbm, out_hbm, idx_v, acc_v, sem):
    wid = lax.axis_index("s") * _NC + lax.axis_index("c")
    base = wid * _BPW

    def stage(j, carry):
        pltpu.sync_copy(
            textT_hbm.at[j, pl.ds(base, _BPW)],
            idx_v.at[pl.ds(j * _BPW, _BPW)],
        )
        return carry

    lax.fori_loop(0, L, stage, 0)

    pltpu.async_copy(emb_hbm.at[idx_v.at[pl.ds(0, _BPW)]], acc_v, sem).wait()

    def body(j, carry):
        pltpu.async_copy(
            emb_hbm.at[idx_v.at[pl.ds(j * _BPW, _BPW)]], acc_v, sem, add=True
        ).wait()
        return carry

    lax.fori_loop(1, L, body, 0)
    pltpu.sync_copy(acc_v, out_hbm.at[pl.ds(base, _BPW)])


@jax.jit
def _bag(textT, emb):
    mesh = plsc.VectorSubcoreMesh(core_axis_name="c", subcore_axis_name="s")
    return pl.kernel(
        _bag_body,
        out_type=jax.ShapeDtypeStruct((B, EMBED), jnp.float32),
        mesh=mesh,
        scratch_types=[
            pltpu.VMEM((L * _BPW,), jnp.int32),
            pltpu.VMEM((_BPW, EMBED), jnp.float32),
            pltpu.SemaphoreType.DMA,
        ],
        compiler_params=pltpu.CompilerParams(use_tc_tiling_on_sc=False),
    )(textT, emb)


_BLK = 4096


def _mlp_body(x_ref, w1_ref, b1_ref, w2_ref, b2_ref, w3_ref, b3_ref, o_ref):
    x = x_ref[...] * (1.0 / L)
    h = jnp.dot(x, w1_ref[...], preferred_element_type=jnp.float32) + b1_ref[...]
    h = jnp.maximum(h, 0.0)
    h = jnp.dot(h, w2_ref[...], preferred_element_type=jnp.float32) + b2_ref[...]
    h = jnp.maximum(h, 0.0)
    o_ref[...] = (
        jnp.dot(h, w3_ref[...], preferred_element_type=jnp.float32) + b3_ref[...]
    )


@jax.jit
def _mlp(sums, w1t, b1, w2t, b2, w3t, b3):
    grid = (B // _BLK,)
    return pl.pallas_call(
        _mlp_body,
        grid=grid,
        in_specs=[
            pl.BlockSpec((_BLK, EMBED), lambda i: (i, 0)),
            pl.BlockSpec((EMBED, EMBED), lambda i: (0, 0)),
            pl.BlockSpec((1, EMBED), lambda i: (0, 0)),
            pl.BlockSpec((EMBED, EMBED // 2), lambda i: (0, 0)),
            pl.BlockSpec((1, EMBED // 2), lambda i: (0, 0)),
            pl.BlockSpec((EMBED // 2, 1), lambda i: (0, 0)),
            pl.BlockSpec((1, 1), lambda i: (0, 0)),
        ],
        out_specs=pl.BlockSpec((_BLK, 1), lambda i: (i, 0)),
        out_shape=jax.ShapeDtypeStruct((B, 1), jnp.float32),
    )(sums, w1t, b1, w2t, b2, w3t, b3)


def kernel(text, emb, W1, b1, W2, b2, W3, b3):
    textT = jnp.transpose(text).astype(jnp.int32)
    sums = _bag(textT, emb)
    out = _mlp(
        sums,
        jnp.transpose(W1),
        b1.reshape(1, EMBED),
        jnp.transpose(W2),
        b2.reshape(1, EMBED // 2),
        jnp.transpose(W3),
        b3.reshape(1, 1),
    )
    return jnp.squeeze(out, axis=-1)


# speedup vs baseline: 2.7075x; 1.0175x over previous
import functools

import jax
import jax.numpy as jnp
from jax import lax
from jax.experimental import pallas as pl
from jax.experimental.pallas import tpu as pltpu
from jax.experimental.pallas import tpu_sc as plsc

VOCAB = 1000000
EMBED = 64
B = 16384
L = 50

try:
    _info = plsc.get_sparse_core_info()
    _NC, _NS = _info.num_cores, _info.num_subcores
except Exception:
    _NC, _NS = 2, 16
_NW = _NC * _NS
_BPW = B // _NW


def _bag_body(text_hbm, emb_hbm, out_hbm, raw_v, idx_v, acc_v, sem):
    wid = lax.axis_index("s") * _NC + lax.axis_index("c")
    base = wid * _BPW

    pltpu.sync_copy(text_hbm.at[pl.ds(base * L, _BPW * L)], raw_v)

    lanes = lax.iota(jnp.int32, 16) * L

    def transp_j(j, carry):
        def transp_g(g, carry2):
            src = lanes + (g * (16 * L) + j)
            vals = plsc.load_gather(raw_v, [src])
            idx_v[pl.ds(j * _BPW + g * 16, 16)] = vals
            return carry2

        lax.fori_loop(0, _BPW // 16, transp_g, 0)
        return carry

    lax.fori_loop(0, L, transp_j, 0)

    pltpu.async_copy(emb_hbm.at[idx_v.at[pl.ds(0, _BPW)]], acc_v, sem).wait()

    def body(j, carry):
        pltpu.async_copy(
            emb_hbm.at[idx_v.at[pl.ds(j * _BPW, _BPW)]], acc_v, sem, add=True
        ).wait()
        return carry

    lax.fori_loop(1, L, body, 0)
    pltpu.sync_copy(acc_v, out_hbm.at[pl.ds(base, _BPW)])


@jax.jit
def _bag(text_flat, emb):
    mesh = plsc.VectorSubcoreMesh(core_axis_name="c", subcore_axis_name="s")
    return pl.kernel(
        _bag_body,
        out_type=jax.ShapeDtypeStruct((B, EMBED), jnp.float32),
        mesh=mesh,
        scratch_types=[
            pltpu.VMEM((_BPW * L,), jnp.int32),
            pltpu.VMEM((L * _BPW,), jnp.int32),
            pltpu.VMEM((_BPW, EMBED), jnp.float32),
            pltpu.SemaphoreType.DMA,
        ],
        compiler_params=pltpu.CompilerParams(
            use_tc_tiling_on_sc=False, needs_layout_passes=False
        ),
    )(text_flat, emb)


_BLK = 4096


def _mlp_body(x_ref, w1_ref, b1_ref, w2_ref, b2_ref, w3_ref, b3_ref, o_ref):
    x = x_ref[...] * (1.0 / L)
    h = jnp.dot(x, w1_ref[...], preferred_element_type=jnp.float32) + b1_ref[...]
    h = jnp.maximum(h, 0.0)
    h = jnp.dot(h, w2_ref[...], preferred_element_type=jnp.float32) + b2_ref[...]
    h = jnp.maximum(h, 0.0)
    o_ref[...] = (
        jnp.dot(h, w3_ref[...], preferred_element_type=jnp.float32) + b3_ref[...]
    )


@jax.jit
def _mlp(sums, w1t, b1, w2t, b2, w3t, b3):
    grid = (B // _BLK,)
    return pl.pallas_call(
        _mlp_body,
        grid=grid,
        in_specs=[
            pl.BlockSpec((_BLK, EMBED), lambda i: (i, 0)),
            pl.BlockSpec((EMBED, EMBED), lambda i: (0, 0)),
            pl.BlockSpec((1, EMBED), lambda i: (0, 0)),
            pl.BlockSpec((EMBED, EMBED // 2), lambda i: (0, 0)),
            pl.BlockSpec((1, EMBED // 2), lambda i: (0, 0)),
            pl.BlockSpec((EMBED // 2, 1), lambda i: (0, 0)),
            pl.BlockSpec((1, 1), lambda i: (0, 0)),
        ],
        out_specs=pl.BlockSpec((_BLK, 1), lambda i: (i, 0)),
        out_shape=jax.ShapeDtypeStruct((B, 1), jnp.float32),
    )(sums, w1t, b1, w2t, b2, w3t, b3)


def kernel(text, emb, W1, b1, W2, b2, W3, b3):
    text_flat = text.astype(jnp.int32).reshape(-1)
    sums = _bag(text_flat, emb)
    out = _mlp(
        sums,
        jnp.transpose(W1),
        b1.reshape(1, EMBED),
        jnp.transpose(W2),
        b2.reshape(1, EMBED // 2),
        jnp.transpose(W3),
        b3.reshape(1, 1),
    )
    return jnp.squeeze(out, axis=-1)
